# baseline (speedup 1.0000x reference)
import functools

import jax
import jax.numpy as jnp
from jax import lax
from jax.experimental import pallas as pl
from jax.experimental.pallas import tpu as pltpu
from jax.experimental.pallas import tpu_sc as plsc

N = 10000
E = 320000
NFEAT = 128
NHID = 256
NCLASS = 64

NTILE = 16
NPAD = 10240
ROWS_PER_TILE = NPAD // NTILE
CH = 128
EPW = E // 32
NB = 8
NCH1 = 80
DUMP_ROW = N


def _mesh():
    return plsc.VectorSubcoreMesh(core_axis_name="c", subcore_axis_name="s",
                                  num_cores=2, num_subcores=16)


def _zero_rows(rows):
    def zr(i, _):
        rows[i // 8, pl.ds((i % 8) * 16, 16)] = jnp.zeros((16,), jnp.float32)
        return _
    lax.fori_loop(0, CH * 8, zr, None)


def _zero_acc(rows, acc, base):
    def zacc(k, _):
        pltpu.sync_copy(rows, acc.at[pl.ds(base + k * CH, CH)])
        return _
    lax.fori_loop(0, ROWS_PER_TILE // CH, zacc, None)


def _copy_out(acc, rows, base, cid, out0, out1):
    def cp(k, _):
        sl = pl.ds(base + k * CH, CH)
        pltpu.sync_copy(acc.at[sl], rows)

        @pl.when(cid == 0)
        def _():
            pltpu.sync_copy(rows, out0.at[sl])

        @pl.when(cid == 1)
        def _():
            pltpu.sync_copy(rows, out1.at[sl])
        return _
    lax.fori_loop(0, ROWS_PER_TILE // CH, cp, None)


def _make_sc_agg(edge_split):
    nch = NCH1 if edge_split else 2 * NCH1
    out_type = [
        jax.ShapeDtypeStruct((NPAD, 128), jnp.float32),
        jax.ShapeDtypeStruct((NPAD, 128), jnp.float32),
    ]
    scratch = [
        pltpu.VMEM((NB, 2, CH), jnp.int32),
        pltpu.VMEM((NB, 2, CH), jnp.int32),
        pltpu.VMEM((CH, 128), jnp.float32),
        pltpu.VMEM((CH, 128), jnp.float32),
        pltpu.VMEM_SHARED((NPAD, 128), jnp.float32),
        pltpu.SemaphoreType.DMA,
        pltpu.SemaphoreType.DMA,
        pltpu.SemaphoreType.DMA,
        pltpu.SemaphoreType.DMA,
    ]

    def body(table, sdw, out0, out1, ixa, ixb, rows0, rows1, acc,
             sem0, sem1, isema, isemb):
        cid = lax.axis_index("c")
        sid = lax.axis_index("s")
        base = sid * ROWS_PER_TILE

        _zero_rows(rows0)
        _zero_acc(rows0, acc, base)
        plsc.subcore_barrier()

        bufs = ((rows0, sem0), (rows1, sem1))
        ixs = ((ixa, isema), (ixb, isemb))

        def sdw_slice(off):
            sl = pl.ds(off, NB)
            return sdw.at[g_, sl] if edge_split else sdw.at[cid, g_, sl]

        g_ = cid * 16 + sid if edge_split else sid

        def issue_idx(b, q):
            ix, ism = ixs[q]
            off = jnp.minimum(b * NB, nch - NB)
            pltpu.async_copy(sdw_slice(off), ix, ism)

        def wait_idx(q):
            ix, ism = ixs[q]
            pltpu.make_async_copy(sdw_slice(0), ix, ism).wait()

        def issue_g(q, j, p):
            r, sm = bufs[p]
            pltpu.async_copy(table.at[ixs[q][0].at[j, 0]], r, sm)

        def wait_g(p):
            r, sm = bufs[p]
            pltpu.make_async_copy(table.at[ixa.at[0, 0]], r, sm).wait()

        def scat(q, j, p):
            pltpu.sync_copy(bufs[p][0], acc.at[ixs[q][0].at[j, 1]], add=True)

        issue_idx(0, 0)

        def pair(k, _):
            b0 = 2 * k
            wait_idx(0)
            issue_idx(b0 + 1, 1)
            issue_g(0, 0, 0)
            for j in range(1, NB):
                issue_g(0, j, j % 2)
                wait_g((j - 1) % 2)
                scat(0, j - 1, (j - 1) % 2)
            wait_idx(1)
            issue_g(1, 0, 0)
            wait_g((NB - 1) % 2)
            scat(0, NB - 1, (NB - 1) % 2)
            issue_idx(b0 + 2, 0)
            for j in range(1, NB):
                issue_g(1, j, j % 2)
                wait_g((j - 1) % 2)
                scat(1, j - 1, (j - 1) % 2)
            wait_g((NB - 1) % 2)
            scat(1, NB - 1, (NB - 1) % 2)
            return _

        lax.fori_loop(0, nch // (2 * NB), pair, None)
        wait_idx(0)

        plsc.subcore_barrier()
        _copy_out(acc, rows0, base, cid, out0, out1)

    return pl.kernel(body, out_type=out_type, mesh=_mesh(),
                     scratch_types=scratch)


def _make_sc_cnt():
    out_type = [
        jax.ShapeDtypeStruct((NPAD, 128), jnp.float32),
        jax.ShapeDtypeStruct((NPAD, 128), jnp.float32),
    ]
    scratch = [
        pltpu.VMEM((NB, CH), jnp.int32),
        pltpu.VMEM((CH, 128), jnp.float32),
        pltpu.VMEM_SHARED((NPAD, 128), jnp.float32),
        pltpu.SemaphoreType.DMA,
    ]

    def body(dstw, out0, out1, idx_d, rows, acc, sem):
        cid = lax.axis_index("c")
        sid = lax.axis_index("s")
        base = sid * ROWS_PER_TILE
        g = cid * 16 + sid

        _zero_rows(rows)
        _zero_acc(rows, acc, base)

        def fo(i, _):
            rows[i // 8, pl.ds((i % 8) * 16, 16)] = jnp.ones((16,), jnp.float32)
            return _
        lax.fori_loop(0, CH * 8, fo, None)

        plsc.subcore_barrier()

        def blk(b, _):
            pltpu.sync_copy(dstw.at[g, pl.ds(b * NB, NB)], idx_d)
            descs = [pltpu.async_copy(rows, acc.at[idx_d.at[j]], sem,
                                      add=True) for j in range(NB)]
            for dsc in descs:
                dsc.wait()
            return _

        lax.fori_loop(0, NCH1 // NB, blk, None)

        plsc.subcore_barrier()
        _copy_out(acc, rows, base, cid, out0, out1)

    return pl.kernel(body, out_type=out_type, mesh=_mesh(),
                     scratch_types=scratch)


@functools.cache
def _sc_agg(edge_split):
    return _make_sc_agg(edge_split)


@functools.cache
def _sc_cnt():
    return _make_sc_cnt()

BR = 1280
GRID = NPAD // BR


def _dotT(a, w):
    return lax.dot_general(a, w, (((1,), (1,)), ((), ())),
                           preferred_element_type=jnp.float32)


def _b1_body(agg0, agg1, cnt0, cnt1, x, wl1, wr1, bl1, t_out, stats_out, acc):
    mean = agg0[...] + agg1[...]
    mean = mean / jnp.maximum(cnt0[:, 0:1] + cnt1[:, 0:1], 1.0)
    out = _dotT(mean, wl1[...]) + bl1[...] + _dotT(x[...], wr1[...])
    nrm = jnp.sqrt(jnp.sum(out * out, axis=1, keepdims=True))
    out = out / jnp.maximum(nrm, 1e-12)
    out = jnp.maximum(out, 0.0)
    t_out[...] = out
    m = pl.program_id(0)
    rowid = m * BR + lax.broadcasted_iota(jnp.int32, (BR, NHID), 0)
    o2 = jnp.where(rowid < N, out, 0.0)
    upd = jnp.concatenate([jnp.sum(o2, axis=0, keepdims=True),
                           jnp.sum(o2 * o2, axis=0, keepdims=True)], axis=0)

    @pl.when(m == 0)
    def _():
        acc[...] = upd

    @pl.when(m > 0)
    def _():
        acc[...] = acc[...] + upd

    @pl.when(m == GRID - 1)
    def _():
        tot = acc[...]
        mu = tot[0:1, :] / float(N)
        var = tot[1:2, :] / float(N) - mu * mu
        stats_out[...] = jnp.concatenate([mu, var], axis=0)


_b1 = pl.pallas_call(
    _b1_body,
    grid=(GRID,),
    in_specs=[
        pl.BlockSpec((BR, 128), lambda m: (m, 0)),
        pl.BlockSpec((BR, 128), lambda m: (m, 0)),
        pl.BlockSpec((BR, 128), lambda m: (m, 0)),
        pl.BlockSpec((BR, 128), lambda m: (m, 0)),
        pl.BlockSpec((BR, NFEAT), lambda m: (m, 0)),
        pl.BlockSpec((NHID, NFEAT), lambda m: (0, 0)),
        pl.BlockSpec((NHID, NFEAT), lambda m: (0, 0)),
        pl.BlockSpec((1, NHID), lambda m: (0, 0)),
    ],
    out_specs=[
        pl.BlockSpec((BR, NHID), lambda m: (m, 0)),
        pl.BlockSpec((2, NHID), lambda m: (0, 0)),
    ],
    out_shape=[
        jax.ShapeDtypeStruct((NPAD, NHID), jnp.float32),
        jax.ShapeDtypeStruct((2, NHID), jnp.float32),
    ],
    scratch_shapes=[pltpu.VMEM((2, NHID), jnp.float32)],
)


def _b2_body(t, stats, gamma, beta, wr2, bl2, hlo, hhi, y2):
    mu = stats[0:1, :]
    var = stats[1:2, :]
    h = gamma[...] * (t[...] - mu) / jnp.sqrt(var + 1e-5) + beta[...]
    y2[...] = _dotT(h, wr2[...]) + bl2[...]
    hlo[...] = h[:, :128]
    hhi[...] = h[:, 128:]


_b2 = pl.pallas_call(
    _b2_body,
    grid=(GRID,),
    in_specs=[
        pl.BlockSpec((BR, NHID), lambda m: (m, 0)),
        pl.BlockSpec((2, NHID), lambda m: (0, 0)),
        pl.BlockSpec((1, NHID), lambda m: (0, 0)),
        pl.BlockSpec((1, NHID), lambda m: (0, 0)),
        pl.BlockSpec((NHID, NHID), lambda m: (0, 0)),
        pl.BlockSpec((1, NHID), lambda m: (0, 0)),
    ],
    out_specs=[
        pl.BlockSpec((BR, 128), lambda m: (m, 0)),
        pl.BlockSpec((BR, 128), lambda m: (m, 0)),
        pl.BlockSpec((BR, NHID), lambda m: (m, 0)),
    ],
    out_shape=[
        jax.ShapeDtypeStruct((NPAD, 128), jnp.float32),
        jax.ShapeDtypeStruct((NPAD, 128), jnp.float32),
        jax.ShapeDtypeStruct((NPAD, NHID), jnp.float32),
    ],
)


def _d_body(a2lo, a2hi, cnt0, cnt1, y2, wl2, wlin, blin, o):
    mean = jnp.concatenate([a2lo[...], a2hi[...]], axis=1)
    mean = mean / jnp.maximum(cnt0[:, 0:1] + cnt1[:, 0:1], 1.0)
    h = _dotT(mean, wl2[...]) + y2[...]
    nrm = jnp.sqrt(jnp.sum(h * h, axis=1, keepdims=True))
    h = h / jnp.maximum(nrm, 1e-12)
    o[...] = _dotT(h, wlin[...]) + blin[...]


_d = pl.pallas_call(
    _d_body,
    grid=(GRID,),
    in_specs=[
        pl.BlockSpec((BR, 128), lambda m: (m, 0)),
        pl.BlockSpec((BR, 128), lambda m: (m, 0)),
        pl.BlockSpec((BR, 128), lambda m: (m, 0)),
        pl.BlockSpec((BR, 128), lambda m: (m, 0)),
        pl.BlockSpec((BR, NHID), lambda m: (m, 0)),
        pl.BlockSpec((NHID, NHID), lambda m: (0, 0)),
        pl.BlockSpec((NCLASS, NHID), lambda m: (0, 0)),
        pl.BlockSpec((1, NCLASS), lambda m: (0, 0)),
    ],
    out_specs=[pl.BlockSpec((BR, NCLASS), lambda m: (m, 0))],
    out_shape=[jax.ShapeDtypeStruct((NPAD, NCLASS), jnp.float32)],
)


def kernel(x, edge_index, Wl1, bl1, Wr1, gamma, beta, Wl2, bl2, Wr2, Wlin, blin):
    src = edge_index[0]
    dst = edge_index[1]
    pad = NCH1 * CH - EPW
    src_p = jnp.concatenate(
        [src.reshape(32, EPW), jnp.zeros((32, pad), jnp.int32)],
        axis=1).reshape(32, NCH1, CH)
    dst_p = jnp.concatenate(
        [dst.reshape(32, EPW), jnp.full((32, pad), DUMP_ROW, jnp.int32)],
        axis=1).reshape(32, NCH1, CH)

    sdw1 = jnp.stack([src_p, dst_p], axis=2)
    s2 = jnp.concatenate([src_p[:16], src_p[16:]], axis=1)
    d2 = jnp.concatenate([dst_p[:16], dst_p[16:]], axis=1)
    sdw2 = jnp.stack([jnp.stack([s2, d2], axis=2),
                      jnp.stack([s2 + NPAD, d2], axis=2)], axis=0)

    x_pad = jnp.pad(x, ((0, NPAD - N), (0, 0)))

    cnt0, cnt1 = _sc_cnt()(dst_p)
    agg0, agg1 = _sc_agg(True)(x, sdw1)
    t, stats = _b1(agg0, agg1, cnt0, cnt1, x_pad, Wl1, Wr1,
                   bl1.reshape(1, NHID))
    hlo, hhi, y2 = _b2(t, stats, gamma.reshape(1, NHID),
                       beta.reshape(1, NHID), Wr2, bl2.reshape(1, NHID))
    hcat = jnp.concatenate([hlo, hhi], axis=0)
    a2lo, a2hi = _sc_agg(False)(hcat, sdw2)
    (o,) = _d(a2lo, a2hi, cnt0, cnt1, y2, Wl2, Wlin, blin.reshape(1, NCLASS))
    return o[:N]


# speedup vs baseline: 1.1323x; 1.1323x over previous
import functools

import jax
import jax.numpy as jnp
from jax import lax
from jax.experimental import pallas as pl
from jax.experimental.pallas import tpu as pltpu
from jax.experimental.pallas import tpu_sc as plsc

N = 10000
E = 320000
NFEAT = 128
NHID = 256
NCLASS = 64

NTILE = 16
NPAD = 10240
ROWS_PER_TILE = NPAD // NTILE
CH = 128
EPW = E // 32
NB = 8
NCH1 = 80
DUMP_ROW = N


def _mesh():
    return plsc.VectorSubcoreMesh(core_axis_name="c", subcore_axis_name="s",
                                  num_cores=2, num_subcores=16)


def _zero_rows(rows):
    def zr(i, _):
        rows[i // 8, pl.ds((i % 8) * 16, 16)] = jnp.zeros((16,), jnp.float32)
        return _
    lax.fori_loop(0, CH * 8, zr, None)


def _zero_acc(rows, acc, base):
    def zacc(k, _):
        pltpu.sync_copy(rows, acc.at[pl.ds(base + k * CH, CH)])
        return _
    lax.fori_loop(0, ROWS_PER_TILE // CH, zacc, None)


def _copy_out(acc, rows, base, cid, out0, out1):
    def cp(k, _):
        sl = pl.ds(base + k * CH, CH)
        pltpu.sync_copy(acc.at[sl], rows)

        @pl.when(cid == 0)
        def _():
            pltpu.sync_copy(rows, out0.at[sl])

        @pl.when(cid == 1)
        def _():
            pltpu.sync_copy(rows, out1.at[sl])
        return _
    lax.fori_loop(0, ROWS_PER_TILE // CH, cp, None)


def _make_sc_agg(edge_split):
    nch = NCH1 if edge_split else 2 * NCH1
    out_type = [
        jax.ShapeDtypeStruct((NPAD, 128), jnp.float32),
        jax.ShapeDtypeStruct((NPAD, 128), jnp.float32),
    ]
    scratch = [
        pltpu.VMEM((NB, 2, CH), jnp.int32),
        pltpu.VMEM((NB, 2, CH), jnp.int32),
        pltpu.VMEM((CH, 128), jnp.float32),
        pltpu.VMEM((CH, 128), jnp.float32),
        pltpu.VMEM_SHARED((NPAD, 128), jnp.float32),
        pltpu.SemaphoreType.DMA,
        pltpu.SemaphoreType.DMA,
        pltpu.SemaphoreType.DMA,
        pltpu.SemaphoreType.DMA,
    ]

    def body(table, sdw, out0, out1, ixa, ixb, rows0, rows1, acc,
             sem0, sem1, isema, isemb):
        cid = lax.axis_index("c")
        sid = lax.axis_index("s")
        base = sid * ROWS_PER_TILE

        _zero_rows(rows0)
        _zero_acc(rows0, acc, base)
        plsc.subcore_barrier()

        bufs = ((rows0, sem0), (rows1, sem1))
        ixs = ((ixa, isema), (ixb, isemb))

        def sdw_slice(off):
            sl = pl.ds(off, NB)
            return sdw.at[g_, sl] if edge_split else sdw.at[cid, g_, sl]

        g_ = cid * 16 + sid if edge_split else sid

        def issue_idx(b, q):
            ix, ism = ixs[q]
            off = jnp.minimum(b * NB, nch - NB)
            pltpu.async_copy(sdw_slice(off), ix, ism)

        def wait_idx(q):
            ix, ism = ixs[q]
            pltpu.make_async_copy(sdw_slice(0), ix, ism).wait()

        def issue_g(q, j, p):
            r, sm = bufs[p]
            pltpu.async_copy(table.at[ixs[q][0].at[j, 0]], r, sm)

        def wait_g(p):
            r, sm = bufs[p]
            pltpu.make_async_copy(table.at[ixa.at[0, 0]], r, sm).wait()

        def scat(q, j, p):
            pltpu.sync_copy(bufs[p][0], acc.at[ixs[q][0].at[j, 1]], add=True)

        issue_idx(0, 0)

        def pair(k, _):
            b0 = 2 * k
            wait_idx(0)
            issue_idx(b0 + 1, 1)
            issue_g(0, 0, 0)
            for j in range(1, NB):
                issue_g(0, j, j % 2)
                wait_g((j - 1) % 2)
                scat(0, j - 1, (j - 1) % 2)
            wait_idx(1)
            issue_g(1, 0, 0)
            wait_g((NB - 1) % 2)
            scat(0, NB - 1, (NB - 1) % 2)
            issue_idx(b0 + 2, 0)
            for j in range(1, NB):
                issue_g(1, j, j % 2)
                wait_g((j - 1) % 2)
                scat(1, j - 1, (j - 1) % 2)
            wait_g((NB - 1) % 2)
            scat(1, NB - 1, (NB - 1) % 2)
            return _

        lax.fori_loop(0, nch // (2 * NB), pair, None)
        wait_idx(0)

        plsc.subcore_barrier()
        _copy_out(acc, rows0, base, cid, out0, out1)

    return pl.kernel(body, out_type=out_type, mesh=_mesh(),
                     scratch_types=scratch)


def _make_sc_cnt():
    out_type = [
        jax.ShapeDtypeStruct((NPAD, 128), jnp.float32),
        jax.ShapeDtypeStruct((NPAD, 128), jnp.float32),
    ]
    scratch = [
        pltpu.VMEM((NB, CH), jnp.int32),
        pltpu.VMEM((CH, 128), jnp.float32),
        pltpu.VMEM_SHARED((NPAD, 128), jnp.float32),
        pltpu.SemaphoreType.DMA,
    ]

    def body(dstw, out0, out1, idx_d, rows, acc, sem):
        cid = lax.axis_index("c")
        sid = lax.axis_index("s")
        base = sid * ROWS_PER_TILE
        g = cid * 16 + sid

        _zero_rows(rows)
        _zero_acc(rows, acc, base)

        def fo(i, _):
            rows[i // 8, pl.ds((i % 8) * 16, 16)] = jnp.ones((16,), jnp.float32)
            return _
        lax.fori_loop(0, CH * 8, fo, None)

        plsc.subcore_barrier()

        def blk(b, _):
            pltpu.sync_copy(dstw.at[g, pl.ds(b * NB, NB)], idx_d)
            descs = [pltpu.async_copy(rows, acc.at[idx_d.at[j]], sem,
                                      add=True) for j in range(NB)]
            for dsc in descs:
                dsc.wait()
            return _

        lax.fori_loop(0, NCH1 // NB, blk, None)

        plsc.subcore_barrier()
        _copy_out(acc, rows, base, cid, out0, out1)

    return pl.kernel(body, out_type=out_type, mesh=_mesh(),
                     scratch_types=scratch)


@functools.cache
def _sc_agg(edge_split):
    return _make_sc_agg(edge_split)


@functools.cache
def _sc_cnt():
    return _make_sc_cnt()

BR = 1280
GRID = NPAD // BR


def _dotT(a, w):
    return lax.dot_general(a, w, (((1,), (1,)), ((), ())),
                           preferred_element_type=jnp.float32)


def _b1_body(agg0, agg1, cnt0, cnt1, x, wl1, wr1, bl1, t_out, stats_out, acc):
    mean = agg0[...] + agg1[...]
    mean = mean / jnp.maximum(cnt0[:, 0:1] + cnt1[:, 0:1], 1.0)
    out = _dotT(mean, wl1[...]) + bl1[...] + _dotT(x[...], wr1[...])
    nrm = jnp.sqrt(jnp.sum(out * out, axis=1, keepdims=True))
    out = out / jnp.maximum(nrm, 1e-12)
    out = jnp.maximum(out, 0.0)
    t_out[...] = out
    m = pl.program_id(0)
    rowid = m * BR + lax.broadcasted_iota(jnp.int32, (BR, NHID), 0)
    o2 = jnp.where(rowid < N, out, 0.0)
    upd = jnp.concatenate([jnp.sum(o2, axis=0, keepdims=True),
                           jnp.sum(o2 * o2, axis=0, keepdims=True)], axis=0)

    @pl.when(m == 0)
    def _():
        acc[...] = upd

    @pl.when(m > 0)
    def _():
        acc[...] = acc[...] + upd

    @pl.when(m == GRID - 1)
    def _():
        tot = acc[...]
        mu = tot[0:1, :] / float(N)
        var = tot[1:2, :] / float(N) - mu * mu
        stats_out[...] = jnp.concatenate([mu, var], axis=0)


_b1 = pl.pallas_call(
    _b1_body,
    grid=(GRID,),
    in_specs=[
        pl.BlockSpec((BR, 128), lambda m: (m, 0)),
        pl.BlockSpec((BR, 128), lambda m: (m, 0)),
        pl.BlockSpec((BR, 128), lambda m: (m, 0)),
        pl.BlockSpec((BR, 128), lambda m: (m, 0)),
        pl.BlockSpec((BR, NFEAT), lambda m: (m, 0)),
        pl.BlockSpec((NHID, NFEAT), lambda m: (0, 0)),
        pl.BlockSpec((NHID, NFEAT), lambda m: (0, 0)),
        pl.BlockSpec((1, NHID), lambda m: (0, 0)),
    ],
    out_specs=[
        pl.BlockSpec((BR, NHID), lambda m: (m, 0)),
        pl.BlockSpec((2, NHID), lambda m: (0, 0)),
    ],
    out_shape=[
        jax.ShapeDtypeStruct((NPAD, NHID), jnp.float32),
        jax.ShapeDtypeStruct((2, NHID), jnp.float32),
    ],
    scratch_shapes=[pltpu.VMEM((2, NHID), jnp.float32)],
)


def _b2_body(t, stats, gamma, beta, wr2, bl2, hlo, hhi, y2):
    mu = stats[0:1, :]
    var = stats[1:2, :]
    h = gamma[...] * (t[...] - mu) / jnp.sqrt(var + 1e-5) + beta[...]
    y2[...] = _dotT(h, wr2[...]) + bl2[...]
    hlo[...] = h[:, :128]
    hhi[...] = h[:, 128:]


_b2 = pl.pallas_call(
    _b2_body,
    grid=(GRID,),
    in_specs=[
        pl.BlockSpec((BR, NHID), lambda m: (m, 0)),
        pl.BlockSpec((2, NHID), lambda m: (0, 0)),
        pl.BlockSpec((1, NHID), lambda m: (0, 0)),
        pl.BlockSpec((1, NHID), lambda m: (0, 0)),
        pl.BlockSpec((NHID, NHID), lambda m: (0, 0)),
        pl.BlockSpec((1, NHID), lambda m: (0, 0)),
    ],
    out_specs=[
        pl.BlockSpec((BR, 128), lambda m: (m, 0)),
        pl.BlockSpec((BR, 128), lambda m: (m, 0)),
        pl.BlockSpec((BR, NHID), lambda m: (m, 0)),
    ],
    out_shape=[
        jax.ShapeDtypeStruct((NPAD, 128), jnp.float32),
        jax.ShapeDtypeStruct((NPAD, 128), jnp.float32),
        jax.ShapeDtypeStruct((NPAD, NHID), jnp.float32),
    ],
)


def _d_body(a2lo, a2hi, cnt0, cnt1, y2, wl2, wlin, blin, o):
    mean = jnp.concatenate([a2lo[...], a2hi[...]], axis=1)
    mean = mean / jnp.maximum(cnt0[:, 0:1] + cnt1[:, 0:1], 1.0)
    h = _dotT(mean, wl2[...]) + y2[...]
    nrm = jnp.sqrt(jnp.sum(h * h, axis=1, keepdims=True))
    h = h / jnp.maximum(nrm, 1e-12)
    o[...] = _dotT(h, wlin[...]) + blin[...]


_d = pl.pallas_call(
    _d_body,
    grid=(GRID,),
    in_specs=[
        pl.BlockSpec((BR, 128), lambda m: (m, 0)),
        pl.BlockSpec((BR, 128), lambda m: (m, 0)),
        pl.BlockSpec((BR, 128), lambda m: (m, 0)),
        pl.BlockSpec((BR, 128), lambda m: (m, 0)),
        pl.BlockSpec((BR, NHID), lambda m: (m, 0)),
        pl.BlockSpec((NHID, NHID), lambda m: (0, 0)),
        pl.BlockSpec((NCLASS, NHID), lambda m: (0, 0)),
        pl.BlockSpec((1, NCLASS), lambda m: (0, 0)),
    ],
    out_specs=[pl.BlockSpec((BR, NCLASS), lambda m: (m, 0))],
    out_shape=[jax.ShapeDtypeStruct((NPAD, NCLASS), jnp.float32)],
)


def kernel(x, edge_index, Wl1, bl1, Wr1, gamma, beta, Wl2, bl2, Wr2, Wlin, blin):
    src = edge_index[0]
    dst = edge_index[1]
    pad = NCH1 * CH - EPW
    src_p = jnp.concatenate(
        [src.reshape(32, EPW), jnp.zeros((32, pad), jnp.int32)],
        axis=1).reshape(32, NCH1, CH)
    dst_p = jnp.concatenate(
        [dst.reshape(32, EPW), jnp.full((32, pad), DUMP_ROW, jnp.int32)],
        axis=1).reshape(32, NCH1, CH)

    offs = jnp.where(jnp.arange(32) >= 16, N, 0).astype(jnp.int32)
    sdw1 = jnp.stack([src_p + offs[:, None, None], dst_p], axis=2)
    s2 = jnp.concatenate([src_p[:16], src_p[16:]], axis=1)
    d2 = jnp.concatenate([dst_p[:16], dst_p[16:]], axis=1)
    sdw2 = jnp.stack([jnp.stack([s2, d2], axis=2),
                      jnp.stack([s2 + NPAD, d2], axis=2)], axis=0)

    x_pad = jnp.pad(x, ((0, NPAD - N), (0, 0)))

    cnt0, cnt1 = _sc_cnt()(dst_p)
    xcat = jnp.concatenate([x, x], axis=0)
    agg0, agg1 = _sc_agg(True)(xcat, sdw1)
    t, stats = _b1(agg0, agg1, cnt0, cnt1, x_pad, Wl1, Wr1,
                   bl1.reshape(1, NHID))
    hlo, hhi, y2 = _b2(t, stats, gamma.reshape(1, NHID),
                       beta.reshape(1, NHID), Wr2, bl2.reshape(1, NHID))
    hcat = jnp.concatenate([hlo, hhi], axis=0)
    a2lo, a2hi = _sc_agg(False)(hcat, sdw2)
    (o,) = _d(a2lo, a2hi, cnt0, cnt1, y2, Wl2, Wlin, blin.reshape(1, NCLASS))
    return o[:N]


# speedup vs baseline: 1.1450x; 1.0112x over previous
import functools

import jax
import jax.numpy as jnp
from jax import lax
from jax.experimental import pallas as pl
from jax.experimental.pallas import tpu as pltpu
from jax.experimental.pallas import tpu_sc as plsc

N = 10000
E = 320000
NFEAT = 128
NHID = 256
NCLASS = 64

NTILE = 16
NPAD = 10240
ROWS_PER_TILE = NPAD // NTILE
CH = 128
EPW = E // 32
NB = 8
NCH1 = 80
DUMP_ROW = N


def _mesh():
    return plsc.VectorSubcoreMesh(core_axis_name="c", subcore_axis_name="s",
                                  num_cores=2, num_subcores=16)


def _zero_rows(rows):
    def zr(i, _):
        rows[i // 8, pl.ds((i % 8) * 16, 16)] = jnp.zeros((16,), jnp.float32)
        return _
    lax.fori_loop(0, CH * 8, zr, None)


def _zero_acc(rows, acc, base):
    def zacc(k, _):
        pltpu.sync_copy(rows, acc.at[pl.ds(base + k * CH, CH)])
        return _
    lax.fori_loop(0, ROWS_PER_TILE // CH, zacc, None)


def _copy_out(acc, rows, base, cid, out0, out1):
    def cp(k, _):
        sl = pl.ds(base + k * CH, CH)
        pltpu.sync_copy(acc.at[sl], rows)

        @pl.when(cid == 0)
        def _():
            pltpu.sync_copy(rows, out0.at[sl])

        @pl.when(cid == 1)
        def _():
            pltpu.sync_copy(rows, out1.at[sl])
        return _
    lax.fori_loop(0, ROWS_PER_TILE // CH, cp, None)


def _make_sc_agg(edge_split, with_cnt=False):
    nch = NCH1 if edge_split else 2 * NCH1
    out_type = [
        jax.ShapeDtypeStruct((NPAD, 128), jnp.float32),
        jax.ShapeDtypeStruct((NPAD, 128), jnp.float32),
    ]
    if with_cnt:
        out_type += [jax.ShapeDtypeStruct((NPAD, 128), jnp.float32),
                     jax.ShapeDtypeStruct((NPAD, 128), jnp.float32)]
    scratch = [
        pltpu.VMEM((NB, 2, CH), jnp.int32),
        pltpu.VMEM((NB, 2, CH), jnp.int32),
        pltpu.VMEM((CH, 128), jnp.float32),
        pltpu.VMEM((CH, 128), jnp.float32),
        pltpu.VMEM_SHARED((NPAD, 128), jnp.float32),
        pltpu.SemaphoreType.DMA,
        pltpu.SemaphoreType.DMA,
        pltpu.SemaphoreType.DMA,
        pltpu.SemaphoreType.DMA,
    ]

    def body(table, sdw, out0, out1, *rest):
        if with_cnt:
            (cnt0, cnt1, ixa, ixb, rows0, rows1, acc,
             sem0, sem1, isema, isemb) = rest
        else:
            ixa, ixb, rows0, rows1, acc, sem0, sem1, isema, isemb = rest
        cid = lax.axis_index("c")
        sid = lax.axis_index("s")
        base = sid * ROWS_PER_TILE

        _zero_rows(rows0)
        _zero_acc(rows0, acc, base)

        g_ = cid * 16 + sid if edge_split else sid

        if with_cnt:
            def fo(i, _):
                rows1[i // 8, pl.ds((i % 8) * 16, 16)] = jnp.ones(
                    (16,), jnp.float32)
                return _
            lax.fori_loop(0, CH * 8, fo, None)
            plsc.subcore_barrier()

            def cblk(b, _):
                pltpu.sync_copy(sdw.at[g_, pl.ds(b * NB, NB)], ixa)
                descs = [pltpu.async_copy(rows1, acc.at[ixa.at[j, 1]],
                                          sem1, add=True) for j in range(NB)]
                for dsc in descs:
                    dsc.wait()
                return _
            lax.fori_loop(0, nch // NB, cblk, None)

            plsc.subcore_barrier()
            _copy_out(acc, rows1, base, cid, cnt0, cnt1)
            plsc.subcore_barrier()
            _zero_acc(rows0, acc, base)

        plsc.subcore_barrier()

        bufs = ((rows0, sem0), (rows1, sem1))
        ixs = ((ixa, isema), (ixb, isemb))

        def sdw_slice(off):
            sl = pl.ds(off, NB)
            return sdw.at[g_, sl] if edge_split else sdw.at[cid, g_, sl]

        def issue_idx(b, q):
            ix, ism = ixs[q]
            off = jnp.minimum(b * NB, nch - NB)
            pltpu.async_copy(sdw_slice(off), ix, ism)

        def wait_idx(q):
            ix, ism = ixs[q]
            pltpu.make_async_copy(sdw_slice(0), ix, ism).wait()

        def issue_g(q, j, p):
            r, sm = bufs[p]
            pltpu.async_copy(table.at[ixs[q][0].at[j, 0]], r, sm)

        def wait_g(p):
            r, sm = bufs[p]
            pltpu.make_async_copy(table.at[ixa.at[0, 0]], r, sm).wait()

        def scat(q, j, p):
            pltpu.sync_copy(bufs[p][0], acc.at[ixs[q][0].at[j, 1]], add=True)

        issue_idx(0, 0)

        def pair(k, _):
            b0 = 2 * k
            wait_idx(0)
            issue_idx(b0 + 1, 1)
            issue_g(0, 0, 0)
            for j in range(1, NB):
                issue_g(0, j, j % 2)
                wait_g((j - 1) % 2)
                scat(0, j - 1, (j - 1) % 2)
            wait_idx(1)
            issue_g(1, 0, 0)
            wait_g((NB - 1) % 2)
            scat(0, NB - 1, (NB - 1) % 2)
            issue_idx(b0 + 2, 0)
            for j in range(1, NB):
                issue_g(1, j, j % 2)
                wait_g((j - 1) % 2)
                scat(1, j - 1, (j - 1) % 2)
            wait_g((NB - 1) % 2)
            scat(1, NB - 1, (NB - 1) % 2)
            return _

        lax.fori_loop(0, nch // (2 * NB), pair, None)
        wait_idx(0)

        plsc.subcore_barrier()
        _copy_out(acc, rows0, base, cid, out0, out1)

    return pl.kernel(body, out_type=out_type, mesh=_mesh(),
                     scratch_types=scratch)


def _make_sc_cnt():
    out_type = [
        jax.ShapeDtypeStruct((NPAD, 128), jnp.float32),
        jax.ShapeDtypeStruct((NPAD, 128), jnp.float32),
    ]
    scratch = [
        pltpu.VMEM((NB, CH), jnp.int32),
        pltpu.VMEM((CH, 128), jnp.float32),
        pltpu.VMEM_SHARED((NPAD, 128), jnp.float32),
        pltpu.SemaphoreType.DMA,
    ]

    def body(dstw, out0, out1, idx_d, rows, acc, sem):
        cid = lax.axis_index("c")
        sid = lax.axis_index("s")
        base = sid * ROWS_PER_TILE
        g = cid * 16 + sid

        _zero_rows(rows)
        _zero_acc(rows, acc, base)

        def fo(i, _):
            rows[i // 8, pl.ds((i % 8) * 16, 16)] = jnp.ones((16,), jnp.float32)
            return _
        lax.fori_loop(0, CH * 8, fo, None)

        plsc.subcore_barrier()

        def blk(b, _):
            pltpu.sync_copy(dstw.at[g, pl.ds(b * NB, NB)], idx_d)
            descs = [pltpu.async_copy(rows, acc.at[idx_d.at[j]], sem,
                                      add=True) for j in range(NB)]
            for dsc in descs:
                dsc.wait()
            return _

        lax.fori_loop(0, NCH1 // NB, blk, None)

        plsc.subcore_barrier()
        _copy_out(acc, rows, base, cid, out0, out1)

    return pl.kernel(body, out_type=out_type, mesh=_mesh(),
                     scratch_types=scratch)


@functools.cache
def _sc_agg(edge_split, with_cnt=False):
    return _make_sc_agg(edge_split, with_cnt)


@functools.cache
def _sc_cnt():
    return _make_sc_cnt()

BR = 1280
GRID = NPAD // BR


def _dotT(a, w):
    return lax.dot_general(a, w, (((1,), (1,)), ((), ())),
                           preferred_element_type=jnp.float32)


def _b1_body(agg0, agg1, cnt0, cnt1, x, wl1, wr1, bl1, t_out, stats_out, acc):
    mean = agg0[...] + agg1[...]
    mean = mean / jnp.maximum(cnt0[:, 0:1] + cnt1[:, 0:1], 1.0)
    out = _dotT(mean, wl1[...]) + bl1[...] + _dotT(x[...], wr1[...])
    nrm = jnp.sqrt(jnp.sum(out * out, axis=1, keepdims=True))
    out = out / jnp.maximum(nrm, 1e-12)
    out = jnp.maximum(out, 0.0)
    t_out[...] = out
    m = pl.program_id(0)
    rowid = m * BR + lax.broadcasted_iota(jnp.int32, (BR, NHID), 0)
    o2 = jnp.where(rowid < N, out, 0.0)
    upd = jnp.concatenate([jnp.sum(o2, axis=0, keepdims=True),
                           jnp.sum(o2 * o2, axis=0, keepdims=True)], axis=0)

    @pl.when(m == 0)
    def _():
        acc[...] = upd

    @pl.when(m > 0)
    def _():
        acc[...] = acc[...] + upd

    @pl.when(m == GRID - 1)
    def _():
        tot = acc[...]
        mu = tot[0:1, :] / float(N)
        var = tot[1:2, :] / float(N) - mu * mu
        stats_out[...] = jnp.concatenate([mu, var], axis=0)


_b1 = pl.pallas_call(
    _b1_body,
    grid=(GRID,),
    in_specs=[
        pl.BlockSpec((BR, 128), lambda m: (m, 0)),
        pl.BlockSpec((BR, 128), lambda m: (m, 0)),
        pl.BlockSpec((BR, 128), lambda m: (m, 0)),
        pl.BlockSpec((BR, 128), lambda m: (m, 0)),
        pl.BlockSpec((BR, NFEAT), lambda m: (m, 0)),
        pl.BlockSpec((NHID, NFEAT), lambda m: (0, 0)),
        pl.BlockSpec((NHID, NFEAT), lambda m: (0, 0)),
        pl.BlockSpec((1, NHID), lambda m: (0, 0)),
    ],
    out_specs=[
        pl.BlockSpec((BR, NHID), lambda m: (m, 0)),
        pl.BlockSpec((2, NHID), lambda m: (0, 0)),
    ],
    out_shape=[
        jax.ShapeDtypeStruct((NPAD, NHID), jnp.float32),
        jax.ShapeDtypeStruct((2, NHID), jnp.float32),
    ],
    scratch_shapes=[pltpu.VMEM((2, NHID), jnp.float32)],
)


def _b2_body(t, stats, gamma, beta, wr2, bl2, hlo, hhi, y2):
    mu = stats[0:1, :]
    var = stats[1:2, :]
    h = gamma[...] * (t[...] - mu) / jnp.sqrt(var + 1e-5) + beta[...]
    y2[...] = _dotT(h, wr2[...]) + bl2[...]
    hlo[...] = h[:, :128]
    hhi[...] = h[:, 128:]


_b2 = pl.pallas_call(
    _b2_body,
    grid=(GRID,),
    in_specs=[
        pl.BlockSpec((BR, NHID), lambda m: (m, 0)),
        pl.BlockSpec((2, NHID), lambda m: (0, 0)),
        pl.BlockSpec((1, NHID), lambda m: (0, 0)),
        pl.BlockSpec((1, NHID), lambda m: (0, 0)),
        pl.BlockSpec((NHID, NHID), lambda m: (0, 0)),
        pl.BlockSpec((1, NHID), lambda m: (0, 0)),
    ],
    out_specs=[
        pl.BlockSpec((BR, 128), lambda m: (m, 0)),
        pl.BlockSpec((BR, 128), lambda m: (m, 0)),
        pl.BlockSpec((BR, NHID), lambda m: (m, 0)),
    ],
    out_shape=[
        jax.ShapeDtypeStruct((NPAD, 128), jnp.float32),
        jax.ShapeDtypeStruct((NPAD, 128), jnp.float32),
        jax.ShapeDtypeStruct((NPAD, NHID), jnp.float32),
    ],
)


def _d_body(a2lo, a2hi, cnt0, cnt1, y2, wl2, wlin, blin, o):
    mean = jnp.concatenate([a2lo[...], a2hi[...]], axis=1)
    mean = mean / jnp.maximum(cnt0[:, 0:1] + cnt1[:, 0:1], 1.0)
    h = _dotT(mean, wl2[...]) + y2[...]
    nrm = jnp.sqrt(jnp.sum(h * h, axis=1, keepdims=True))
    h = h / jnp.maximum(nrm, 1e-12)
    o[...] = _dotT(h, wlin[...]) + blin[...]


_d = pl.pallas_call(
    _d_body,
    grid=(GRID,),
    in_specs=[
        pl.BlockSpec((BR, 128), lambda m: (m, 0)),
        pl.BlockSpec((BR, 128), lambda m: (m, 0)),
        pl.BlockSpec((BR, 128), lambda m: (m, 0)),
        pl.BlockSpec((BR, 128), lambda m: (m, 0)),
        pl.BlockSpec((BR, NHID), lambda m: (m, 0)),
        pl.BlockSpec((NHID, NHID), lambda m: (0, 0)),
        pl.BlockSpec((NCLASS, NHID), lambda m: (0, 0)),
        pl.BlockSpec((1, NCLASS), lambda m: (0, 0)),
    ],
    out_specs=[pl.BlockSpec((BR, NCLASS), lambda m: (m, 0))],
    out_shape=[jax.ShapeDtypeStruct((NPAD, NCLASS), jnp.float32)],
)


def kernel(x, edge_index, Wl1, bl1, Wr1, gamma, beta, Wl2, bl2, Wr2, Wlin, blin):
    src = edge_index[0]
    dst = edge_index[1]
    pad = NCH1 * CH - EPW
    src_p = jnp.concatenate(
        [src.reshape(32, EPW), jnp.zeros((32, pad), jnp.int32)],
        axis=1).reshape(32, NCH1, CH)
    dst_p = jnp.concatenate(
        [dst.reshape(32, EPW), jnp.full((32, pad), DUMP_ROW, jnp.int32)],
        axis=1).reshape(32, NCH1, CH)

    offs = jnp.where(jnp.arange(32) >= 16, N, 0).astype(jnp.int32)
    sdw1 = jnp.stack([src_p + offs[:, None, None], dst_p], axis=2)
    s2 = jnp.concatenate([src_p[:16], src_p[16:]], axis=1)
    d2 = jnp.concatenate([dst_p[:16], dst_p[16:]], axis=1)
    sdw2 = jnp.stack([jnp.stack([s2, d2], axis=2),
                      jnp.stack([s2 + NPAD, d2], axis=2)], axis=0)

    x_pad = jnp.pad(x, ((0, NPAD - N), (0, 0)))

    xcat = jnp.concatenate([x, x], axis=0)
    agg0, agg1, cnt0, cnt1 = _sc_agg(True, True)(xcat, sdw1)
    t, stats = _b1(agg0, agg1, cnt0, cnt1, x_pad, Wl1, Wr1,
                   bl1.reshape(1, NHID))
    hlo, hhi, y2 = _b2(t, stats, gamma.reshape(1, NHID),
                       beta.reshape(1, NHID), Wr2, bl2.reshape(1, NHID))
    hcat = jnp.concatenate([hlo, hhi], axis=0)
    a2lo, a2hi = _sc_agg(False)(hcat, sdw2)
    (o,) = _d(a2lo, a2hi, cnt0, cnt1, y2, Wl2, Wlin, blin.reshape(1, NCLASS))
    return o[:N]
